# baseline (speedup 1.0000x reference)
import jax
import jax.numpy as jnp
import numpy as np
from jax.experimental import pallas as pl

D = 768
H = 8
K = 5
L = 3
S = 2048
DH = D // H
DP = 128

NEG = -1e30


def _simtopk_body(xi_ref, xall_ref, e_ref):
    xi = xi_ref[...]
    sim = jnp.dot(xi, xall_ref[...].T, preferred_element_type=jnp.float32)
    bs = sim.shape[0]
    jcol = jax.lax.broadcasted_iota(jnp.int32, (bs, S), 1)
    sel = jnp.zeros_like(sim)
    for _ in range(K):
        v = jnp.max(sim, axis=1, keepdims=True)
        eq = sim >= v
        am = jnp.min(jnp.where(eq, jcol, S), axis=1, keepdims=True)
        onehot = (jcol == am).astype(jnp.float32)
        sel = sel + onehot
        sim = sim + onehot * NEG
    e_ref[...] = sel.astype(jnp.bfloat16)


def _simtopk(x, bs=512):
    return pl.pallas_call(
        _simtopk_body,
        grid=(S // bs,),
        in_specs=[
            pl.BlockSpec((bs, D), lambda i: (i, 0)),
            pl.BlockSpec((S, D), lambda i: (0, 0)),
        ],
        out_specs=pl.BlockSpec((bs, S), lambda i: (i, 0)),
        out_shape=jax.ShapeDtypeStruct((S, S), jnp.bfloat16),
    )(x, x)


def _adjdeg_body(erow_ref, ecol_ref, a_ref, d_ref):
    bs = erow_ref.shape[0]
    i = pl.program_id(0)
    a = jnp.maximum(erow_ref[...], ecol_ref[...].T)
    rid = i * bs + jax.lax.broadcasted_iota(jnp.int32, (bs, S), 0)
    cid = jax.lax.broadcasted_iota(jnp.int32, (bs, S), 1)
    a = jnp.maximum(a, (rid == cid).astype(jnp.bfloat16))
    a_ref[...] = a
    deg = jnp.sum(a.astype(jnp.float32), axis=1, keepdims=True)
    d_ref[...] = 1.0 / jnp.sqrt(deg)


def _adjdeg(e, bs=512):
    return pl.pallas_call(
        _adjdeg_body,
        grid=(S // bs,),
        in_specs=[
            pl.BlockSpec((bs, S), lambda i: (i, 0)),
            pl.BlockSpec((S, bs), lambda i: (0, i)),
        ],
        out_specs=[
            pl.BlockSpec((bs, S), lambda i: (i, 0)),
            pl.BlockSpec((bs, 1), lambda i: (i, 0)),
        ],
        out_shape=[
            jax.ShapeDtypeStruct((S, S), jnp.bfloat16),
            jax.ShapeDtypeStruct((S, 1), jnp.float32),
        ],
    )(e, e)


def _gcnqkv_body(x_ref, d_ref, gw_ref, qw_ref, qb_ref, g_ref, qkv_ref):
    x = x_ref[...]
    xd = x * d_ref[...]
    g = jnp.dot(xd, gw_ref[...].T, preferred_element_type=jnp.float32)
    g_ref[...] = g.astype(jnp.bfloat16)
    qkv_ref[...] = (
        jnp.dot(x, qw_ref[...].T, preferred_element_type=jnp.float32)
        + qb_ref[...]
    )


def _gcnqkv(x, d, gw, qw, qb, bs=512):
    n = qw.shape[0]
    return pl.pallas_call(
        _gcnqkv_body,
        grid=(S // bs,),
        in_specs=[
            pl.BlockSpec((bs, D), lambda i: (i, 0)),
            pl.BlockSpec((bs, 1), lambda i: (i, 0)),
            pl.BlockSpec((D, D), lambda i: (0, 0)),
            pl.BlockSpec((n, D), lambda i: (0, 0)),
            pl.BlockSpec((1, n), lambda i: (0, 0)),
        ],
        out_specs=[
            pl.BlockSpec((bs, D), lambda i: (i, 0)),
            pl.BlockSpec((bs, n), lambda i: (i, 0)),
        ],
        out_shape=[
            jax.ShapeDtypeStruct((S, D), jnp.bfloat16),
            jax.ShapeDtypeStruct((S, n), jnp.float32),
        ],
    )(x, d, gw, qw, qb.reshape(1, n))


def _attn_body(q_ref, k_ref, v_ref, w_ref, b_ref, o_ref):
    bq = q_ref.shape[0]
    acc = jnp.zeros((bq, D), jnp.float32) + b_ref[...]
    scale = 1.0 / np.sqrt(DH)
    for h in range(H):
        qh = q_ref[:, h * DP:(h + 1) * DP]
        kh = k_ref[:, h * DP:(h + 1) * DP]
        vh = v_ref[:, h * DP:(h + 1) * DP]
        s = jax.lax.dot_general(qh, kh, (((1,), (1,)), ((), ())),
                                preferred_element_type=jnp.float32) * scale
        m = jnp.max(s, axis=1, keepdims=True)
        p = jnp.exp(s - m)
        p = p / jnp.sum(p, axis=1, keepdims=True)
        oh = jnp.dot(p, vh, preferred_element_type=jnp.float32)
        wh = w_ref[:, h * DP:(h + 1) * DP]
        acc = acc + jax.lax.dot_general(oh, wh, (((1,), (1,)), ((), ())),
                                        preferred_element_type=jnp.float32)
    o_ref[...] = acc


def _attention(qkv, w, b, bq=512):
    hd = H * DP
    return pl.pallas_call(
        _attn_body,
        grid=(S // bq,),
        in_specs=[
            pl.BlockSpec((bq, hd), lambda i: (i, 0)),
            pl.BlockSpec((S, hd), lambda i: (0, 1)),
            pl.BlockSpec((S, hd), lambda i: (0, 2)),
            pl.BlockSpec((D, hd), lambda i: (0, 0)),
            pl.BlockSpec((1, D), lambda i: (0, 0)),
        ],
        out_specs=pl.BlockSpec((bq, D), lambda i: (i, 0)),
        out_shape=jax.ShapeDtypeStruct((S, D), jnp.float32),
    )(qkv, qkv, qkv, w, b.reshape(1, D))


def _fuse_core(a_ref, g_ref, d_ref, b_ref, attn_ref, xs_ref, gw_ref, gb_ref,
               ng_ref, nb_ref):
    agg = jnp.dot(a_ref[...], g_ref[...], preferred_element_type=jnp.float32)
    gcn = agg * d_ref[...] + b_ref[...]
    attn = attn_ref[...]
    gw = gw_ref[...].astype(jnp.bfloat16)
    z = (
        jnp.dot(gcn.astype(jnp.bfloat16), gw[:, :D].T,
                preferred_element_type=jnp.float32)
        + jnp.dot(attn.astype(jnp.bfloat16), gw[:, D:].T,
                  preferred_element_type=jnp.float32)
        + gb_ref[...]
    )
    gate = jax.nn.sigmoid(z)
    y = gate * gcn + (1.0 - gate) * attn + xs_ref[...]
    mu = jnp.mean(y, axis=1, keepdims=True)
    yc = y - mu
    var = jnp.mean(yc * yc, axis=1, keepdims=True)
    return yc / jnp.sqrt(var + 1e-5) * ng_ref[...] + nb_ref[...]


def _fuse_body(a_ref, g_ref, d_ref, b_ref, attn_ref, xs_ref, gw_ref, gb_ref,
               ng_ref, nb_ref, o_ref):
    o_ref[...] = _fuse_core(a_ref, g_ref, d_ref, b_ref, attn_ref, xs_ref,
                            gw_ref, gb_ref, ng_ref, nb_ref)


def _fuse_proj_body(a_ref, g_ref, d_ref, b_ref, attn_ref, xs_ref, gw_ref,
                    gb_ref, ng_ref, nb_ref, pw_ref, pb_ref, o_ref):
    y = _fuse_core(a_ref, g_ref, d_ref, b_ref, attn_ref, xs_ref, gw_ref,
                   gb_ref, ng_ref, nb_ref)
    o_ref[...] = (
        jnp.dot(y, pw_ref[...].T, preferred_element_type=jnp.float32)
        + pb_ref[...]
    )


def _fuse_ln(a, g, d, b, attn, xs, gw, gb, ng, nb, proj=None, bs=1024):
    row = lambda i: (i, 0)
    full = lambda i: (0, 0)
    in_specs = [
        pl.BlockSpec((bs, S), row),
        pl.BlockSpec((S, D), full),
        pl.BlockSpec((bs, 1), row),
        pl.BlockSpec((1, D), full),
        pl.BlockSpec((bs, D), row),
        pl.BlockSpec((bs, D), row),
        pl.BlockSpec((D, 2 * D), full),
        pl.BlockSpec((1, D), full),
        pl.BlockSpec((1, D), full),
        pl.BlockSpec((1, D), full),
    ]
    args = [a, g, d, b.reshape(1, D), attn, xs, gw, gb.reshape(1, D),
            ng.reshape(1, D), nb.reshape(1, D)]
    body = _fuse_body
    if proj is not None:
        pw, pb = proj
        in_specs += [pl.BlockSpec((D, D), full), pl.BlockSpec((1, D), full)]
        args += [pw, pb.reshape(1, D)]
        body = _fuse_proj_body
    return pl.pallas_call(
        body,
        grid=(S // bs,),
        in_specs=in_specs,
        out_specs=pl.BlockSpec((bs, D), row),
        out_shape=jax.ShapeDtypeStruct((S, D), jnp.float32),
    )(*args)


def kernel(x, gcn_W, gcn_b, attn_in_w, attn_in_b, attn_out_w, attn_out_b,
           norm2_g, norm2_b, gate_W, gate_b, proj_W, proj_b):
    xb = x[0]

    in_w_pad = jnp.pad(attn_in_w.reshape(L, 3 * H, DH, D),
                       [(0, 0), (0, 0), (0, DP - DH), (0, 0)]
                       ).reshape(L, 3 * H * DP, D)
    in_b_pad = jnp.pad(attn_in_b.reshape(L, 3 * H, DH),
                       [(0, 0), (0, 0), (0, DP - DH)]).reshape(L, 3 * H * DP)
    out_w_pad = jnp.pad(attn_out_w.reshape(L, D, H, DH),
                        [(0, 0), (0, 0), (0, 0), (0, DP - DH)]
                        ).reshape(L, D, H * DP)

    e = _simtopk(xb)
    a, d = _adjdeg(e)

    xs = xb
    for l in range(L):
        g, qkv = _gcnqkv(xs, d, gcn_W[l], in_w_pad[l], in_b_pad[l])
        attn = _attention(qkv, out_w_pad[l], attn_out_b[l])
        proj = (proj_W, proj_b) if l == L - 1 else None
        xs = _fuse_ln(a, g, d, gcn_b[l], attn, xs, gate_W[l], gate_b[l],
                      norm2_g[l], norm2_b[l], proj=proj)

    return xs[None]


# speedup vs baseline: 1.0346x; 1.0346x over previous
import jax
import jax.numpy as jnp
import numpy as np
from jax.experimental import pallas as pl

D = 768
H = 8
K = 5
L = 3
S = 2048
DH = D // H
DP = 128

NEG = -1e30


def _simtopk_body(xi_ref, xall_ref, e_ref):
    xi = xi_ref[...]
    sim = jnp.dot(xi, xall_ref[...].T, preferred_element_type=jnp.float32)
    bs = sim.shape[0]
    jcol = jax.lax.broadcasted_iota(jnp.int32, (bs, S), 1)
    sel = jnp.zeros_like(sim)
    for _ in range(K):
        v = jnp.max(sim, axis=1, keepdims=True)
        eq = sim >= v
        am = jnp.min(jnp.where(eq, jcol, S), axis=1, keepdims=True)
        onehot = (jcol == am).astype(jnp.float32)
        sel = sel + onehot
        sim = sim + onehot * NEG
    e_ref[...] = sel.astype(jnp.bfloat16)


def _simtopk(x, bs=512):
    return pl.pallas_call(
        _simtopk_body,
        grid=(S // bs,),
        in_specs=[
            pl.BlockSpec((bs, D), lambda i: (i, 0)),
            pl.BlockSpec((S, D), lambda i: (0, 0)),
        ],
        out_specs=pl.BlockSpec((bs, S), lambda i: (i, 0)),
        out_shape=jax.ShapeDtypeStruct((S, S), jnp.bfloat16),
    )(x, x)


def _adjdeg_body(erow_ref, ecol_ref, a_ref, d_ref):
    bs = erow_ref.shape[0]
    i = pl.program_id(0)
    a = jnp.maximum(erow_ref[...], ecol_ref[...].T)
    rid = i * bs + jax.lax.broadcasted_iota(jnp.int32, (bs, S), 0)
    cid = jax.lax.broadcasted_iota(jnp.int32, (bs, S), 1)
    a = jnp.maximum(a, (rid == cid).astype(jnp.bfloat16))
    a_ref[...] = a
    deg = jnp.sum(a.astype(jnp.float32), axis=1, keepdims=True)
    d_ref[...] = 1.0 / jnp.sqrt(deg)


def _adjdeg(e, bs=512):
    return pl.pallas_call(
        _adjdeg_body,
        grid=(S // bs,),
        in_specs=[
            pl.BlockSpec((bs, S), lambda i: (i, 0)),
            pl.BlockSpec((S, bs), lambda i: (0, i)),
        ],
        out_specs=[
            pl.BlockSpec((bs, S), lambda i: (i, 0)),
            pl.BlockSpec((bs, 1), lambda i: (i, 0)),
        ],
        out_shape=[
            jax.ShapeDtypeStruct((S, S), jnp.bfloat16),
            jax.ShapeDtypeStruct((S, 1), jnp.float32),
        ],
    )(e, e)


def _gcnqkv_body(x_ref, d_ref, gw_ref, qw_ref, qb_ref, g_ref, qkv_ref):
    x = x_ref[...]
    xd = x * d_ref[...]
    g = jnp.dot(xd, gw_ref[...].T, preferred_element_type=jnp.float32)
    g_ref[...] = g.astype(jnp.bfloat16)
    qkv_ref[...] = (
        jnp.dot(x, qw_ref[...].T, preferred_element_type=jnp.float32)
        + qb_ref[...]
    )


def _gcnqkv(x, d, gw, qw, qb, bs=512):
    n = qw.shape[0]
    return pl.pallas_call(
        _gcnqkv_body,
        grid=(S // bs,),
        in_specs=[
            pl.BlockSpec((bs, D), lambda i: (i, 0)),
            pl.BlockSpec((bs, 1), lambda i: (i, 0)),
            pl.BlockSpec((D, D), lambda i: (0, 0)),
            pl.BlockSpec((n, D), lambda i: (0, 0)),
            pl.BlockSpec((1, n), lambda i: (0, 0)),
        ],
        out_specs=[
            pl.BlockSpec((bs, D), lambda i: (i, 0)),
            pl.BlockSpec((bs, n), lambda i: (i, 0)),
        ],
        out_shape=[
            jax.ShapeDtypeStruct((S, D), jnp.bfloat16),
            jax.ShapeDtypeStruct((S, n), jnp.float32),
        ],
    )(x, d, gw, qw, qb.reshape(1, n))


def _attn_body(q_ref, k_ref, v_ref, w_ref, b_ref, o_ref):
    bq = q_ref.shape[0]
    acc = jnp.zeros((bq, D), jnp.float32) + b_ref[...]
    scale = 1.0 / np.sqrt(DH)
    for h in range(H):
        qh = q_ref[:, h * DP:(h + 1) * DP]
        kh = k_ref[:, h * DP:(h + 1) * DP]
        vh = v_ref[:, h * DP:(h + 1) * DP]
        s = jax.lax.dot_general(qh, kh, (((1,), (1,)), ((), ())),
                                preferred_element_type=jnp.float32) * scale
        m = jnp.max(s, axis=1, keepdims=True)
        p = jnp.exp(s - m)
        p = p / jnp.sum(p, axis=1, keepdims=True)
        oh = jnp.dot(p, vh, preferred_element_type=jnp.float32)
        wh = w_ref[:, h * DP:(h + 1) * DP]
        acc = acc + jax.lax.dot_general(oh, wh, (((1,), (1,)), ((), ())),
                                        preferred_element_type=jnp.float32)
    o_ref[...] = acc


def _attention(qkv, w, b, bq=512):
    hd = H * DP
    return pl.pallas_call(
        _attn_body,
        grid=(S // bq,),
        in_specs=[
            pl.BlockSpec((bq, hd), lambda i: (i, 0)),
            pl.BlockSpec((S, hd), lambda i: (0, 1)),
            pl.BlockSpec((S, hd), lambda i: (0, 2)),
            pl.BlockSpec((D, hd), lambda i: (0, 0)),
            pl.BlockSpec((1, D), lambda i: (0, 0)),
        ],
        out_specs=pl.BlockSpec((bq, D), lambda i: (i, 0)),
        out_shape=jax.ShapeDtypeStruct((S, D), jnp.float32),
    )(qkv, qkv, qkv, w, b.reshape(1, D))


def _fuse_core(a_ref, g_ref, d_ref, b_ref, attn_ref, xs_ref, gw_ref, gb_ref,
               ng_ref, nb_ref):
    agg = jnp.dot(a_ref[...], g_ref[...], preferred_element_type=jnp.float32)
    gcn = agg * d_ref[...] + b_ref[...]
    attn = attn_ref[...]
    gw = gw_ref[...].astype(jnp.bfloat16)
    z = (
        jnp.dot(gcn.astype(jnp.bfloat16), gw[:, :D].T,
                preferred_element_type=jnp.float32)
        + jnp.dot(attn.astype(jnp.bfloat16), gw[:, D:].T,
                  preferred_element_type=jnp.float32)
        + gb_ref[...]
    )
    gate = jax.nn.sigmoid(z)
    y = gate * gcn + (1.0 - gate) * attn + xs_ref[...]
    mu = jnp.mean(y, axis=1, keepdims=True)
    yc = y - mu
    var = jnp.mean(yc * yc, axis=1, keepdims=True)
    return yc / jnp.sqrt(var + 1e-5) * ng_ref[...] + nb_ref[...]


def _fuse_body(a_ref, g_ref, d_ref, b_ref, attn_ref, xs_ref, gw_ref, gb_ref,
               ng_ref, nb_ref, o_ref):
    o_ref[...] = _fuse_core(a_ref, g_ref, d_ref, b_ref, attn_ref, xs_ref,
                            gw_ref, gb_ref, ng_ref, nb_ref)


def _fuse_proj_body(a_ref, g_ref, d_ref, b_ref, attn_ref, xs_ref, gw_ref,
                    gb_ref, ng_ref, nb_ref, pw_ref, pb_ref, o_ref):
    y = _fuse_core(a_ref, g_ref, d_ref, b_ref, attn_ref, xs_ref, gw_ref,
                   gb_ref, ng_ref, nb_ref)
    o_ref[...] = (
        jnp.dot(y, pw_ref[...].T, preferred_element_type=jnp.float32)
        + pb_ref[...]
    )


def _fuse_ln(a, g, d, b, attn, xs, gw, gb, ng, nb, proj=None, bs=512):
    row = lambda i: (i, 0)
    full = lambda i: (0, 0)
    in_specs = [
        pl.BlockSpec((bs, S), row),
        pl.BlockSpec((S, D), full),
        pl.BlockSpec((bs, 1), row),
        pl.BlockSpec((1, D), full),
        pl.BlockSpec((bs, D), row),
        pl.BlockSpec((bs, D), row),
        pl.BlockSpec((D, 2 * D), full),
        pl.BlockSpec((1, D), full),
        pl.BlockSpec((1, D), full),
        pl.BlockSpec((1, D), full),
    ]
    args = [a, g, d, b.reshape(1, D), attn, xs, gw, gb.reshape(1, D),
            ng.reshape(1, D), nb.reshape(1, D)]
    body = _fuse_body
    if proj is not None:
        pw, pb = proj
        in_specs += [pl.BlockSpec((D, D), full), pl.BlockSpec((1, D), full)]
        args += [pw, pb.reshape(1, D)]
        body = _fuse_proj_body
    return pl.pallas_call(
        body,
        grid=(S // bs,),
        in_specs=in_specs,
        out_specs=pl.BlockSpec((bs, D), row),
        out_shape=jax.ShapeDtypeStruct((S, D), jnp.float32),
    )(*args)


def kernel(x, gcn_W, gcn_b, attn_in_w, attn_in_b, attn_out_w, attn_out_b,
           norm2_g, norm2_b, gate_W, gate_b, proj_W, proj_b):
    xb = x[0]

    in_w_pad = jnp.pad(attn_in_w.reshape(L, 3 * H, DH, D),
                       [(0, 0), (0, 0), (0, DP - DH), (0, 0)]
                       ).reshape(L, 3 * H * DP, D)
    in_b_pad = jnp.pad(attn_in_b.reshape(L, 3 * H, DH),
                       [(0, 0), (0, 0), (0, DP - DH)]).reshape(L, 3 * H * DP)
    out_w_pad = jnp.pad(attn_out_w.reshape(L, D, H, DH),
                        [(0, 0), (0, 0), (0, 0), (0, DP - DH)]
                        ).reshape(L, D, H * DP)

    e = _simtopk(xb)
    a, d = _adjdeg(e)

    xs = xb
    for l in range(L):
        g, qkv = _gcnqkv(xs, d, gcn_W[l], in_w_pad[l], in_b_pad[l])
        attn = _attention(qkv, out_w_pad[l], attn_out_b[l])
        proj = (proj_W, proj_b) if l == L - 1 else None
        xs = _fuse_ln(a, g, d, gcn_b[l], attn, xs, gate_W[l], gate_b[l],
                      norm2_g[l], norm2_b[l], proj=proj)

    return xs[None]
